# baseline (speedup 1.0000x reference)
import functools

import jax
import jax.numpy as jnp
from jax.experimental import pallas as pl
from jax.experimental.pallas import tpu as pltpu

_TILE = 2048
_EPAD = 8
_NEG = -1e30
_INBUF = 4


def _outer_kernel(hw, nt, zc_hbm, zl_hbm, w1c_ref, w1l_ref, b1_ref,
                  w2_ref, b2_ref,
                  oc_hbm, ol_hbm, okeep_hbm, oprobs_hbm, ogate_hbm, oksum_ref,
                  acc_ref):

    def body(zc_ref, zl_ref, oc_ref, ol_ref, okeep_ref, oprobs_ref, ogate_ref):
        i = pl.program_id(0)
        b = i // nt
        t = i % nt
        xc = zc_ref[0]
        xl = zl_ref[0]
        ncols = xc.shape[-1]

        h = (jnp.dot(w1c_ref[...], xc, preferred_element_type=jnp.float32)
             + jnp.dot(w1l_ref[...], xl, preferred_element_type=jnp.float32)
             + b1_ref[...])
        h = h * jax.nn.sigmoid(h)

        logits = jnp.dot(w2_ref[...], h, preferred_element_type=jnp.float32) \
            + b2_ref[...]
        m = jnp.max(logits, axis=0, keepdims=True)
        e = jnp.exp(logits - m)
        p = e / jnp.sum(e, axis=0, keepdims=True)

        amax = jnp.argmax(p, axis=0)
        row = jax.lax.broadcasted_iota(jnp.int32, p.shape, 0)
        g = jnp.where(row == amax[None, :], p, 0.0)

        gc = g[0:1, :]
        gl = g[1:2, :]
        keep = ((gc + gl) > 0).astype(jnp.float32)

        oc_ref[0] = xc * gc
        ol_ref[0] = xl * gl
        okeep_ref[0] = keep
        oprobs_ref[0] = p
        ogate_ref[0] = g

        col = jax.lax.broadcasted_iota(jnp.int32, (1, ncols), 1) + t * ncols
        s = jnp.sum(jnp.where(col < hw, keep, 0.0))
        blk = jnp.full((_EPAD, 128), s, dtype=jnp.float32)

        @pl.when(t == 0)
        def _():
            acc_ref[...] = blk

        @pl.when(t != 0)
        def _():
            acc_ref[...] = acc_ref[...] + blk

        @pl.when(t == nt - 1)
        def _():
            oksum_ref[b] = acc_ref[...]

    bufd = pl.Buffered(buffer_count=_INBUF)
    small = pl.BlockSpec((1, _EPAD, _TILE), lambda i: (i // nt, 0, i % nt))
    one = pl.BlockSpec((1, 1, _TILE), lambda i: (i // nt, 0, i % nt))

    C = zc_hbm.shape[1]
    big_in = pl.BlockSpec((1, C, _TILE), lambda i: (i // nt, 0, i % nt),
                          pipeline_mode=bufd)
    big_out = pl.BlockSpec((1, C, _TILE), lambda i: (i // nt, 0, i % nt))

    pipe = pltpu.emit_pipeline(
        body,
        grid=(zc_hbm.shape[0] * nt,),
        in_specs=[big_in, big_in],
        out_specs=[big_out, big_out, one, small, small],
    )
    pipe(zc_hbm, zl_hbm, oc_hbm, ol_hbm, okeep_hbm, oprobs_hbm, ogate_hbm)


@jax.jit
def kernel(z_cam, z_lidar, W1, b1, W2, b2):
    bsz, C, h, w = z_cam.shape
    hw = h * w
    hidden = W1.shape[1]
    E = W2.shape[1]

    zc = z_cam.reshape(bsz, C, hw)
    zl = z_lidar.reshape(bsz, C, hw)
    w1c = W1[:C].T
    w1l = W1[C:].T
    b1c = b1.reshape(hidden, 1)
    w2p = jnp.zeros((_EPAD, hidden), jnp.float32).at[:E].set(W2.T)
    b2p = jnp.full((_EPAD,), _NEG, jnp.float32).at[:E].set(b2).reshape(_EPAD, 1)

    nt = pl.cdiv(hw, _TILE)

    out_types = (
        jax.ShapeDtypeStruct((bsz, C, hw), jnp.float32),
        jax.ShapeDtypeStruct((bsz, C, hw), jnp.float32),
        jax.ShapeDtypeStruct((bsz, 1, hw), jnp.float32),
        jax.ShapeDtypeStruct((bsz, _EPAD, hw), jnp.float32),
        jax.ShapeDtypeStruct((bsz, _EPAD, hw), jnp.float32),
        jax.ShapeDtypeStruct((bsz, _EPAD, 128), jnp.float32),
    )

    hbm = pl.BlockSpec(memory_space=pltpu.MemorySpace.HBM)
    vmem = pl.BlockSpec(memory_space=pltpu.MemorySpace.VMEM)

    oc, ol, okeep, oprobs, ogate, oksum = pl.pallas_call(
        functools.partial(_outer_kernel, hw, nt),
        in_specs=[hbm, hbm, vmem, vmem, vmem, vmem, vmem],
        out_specs=[hbm, hbm, hbm, hbm, hbm, vmem],
        out_shape=out_types,
        scratch_shapes=[pltpu.VMEM((_EPAD, 128), jnp.float32)],
    )(zc, zl, w1c, w1l, b1c, w2p, b2p)

    zhat_cam = oc.reshape(bsz, C, h, w)
    zhat_lidar = ol.reshape(bsz, C, h, w)
    keep_mask_2d = okeep.reshape(bsz, 1, h, w)
    probs = jnp.transpose(oprobs[:, :E, :], (0, 2, 1))
    gate = jnp.transpose(ogate[:, :E, :], (0, 2, 1))
    keep_ratio = oksum[:, 0:1, 0] / jnp.float32(hw)
    return (zhat_cam, zhat_lidar, keep_mask_2d, probs, gate, keep_ratio)


# speedup vs baseline: 1.0010x; 1.0010x over previous
import functools

import jax
import jax.numpy as jnp
from jax.experimental import pallas as pl
from jax.experimental.pallas import tpu as pltpu

_TILE = 2048
_EPAD = 8
_NEG = -1e30
_INBUF = 3


def _outer_kernel(hw, nt, zc_hbm, zl_hbm, w1c_ref, w1l_ref, b1_ref,
                  w2_ref, b2_ref,
                  oc_hbm, ol_hbm, okeep_ref, oprobs_ref, ogate_ref, oksum_ref,
                  acc_ref):

    def body(zc_ref, zl_ref, oc_ref, ol_ref):
        i = pl.program_id(0)
        b = i // nt
        t = i % nt
        xc = zc_ref[0]
        xl = zl_ref[0]
        ncols = xc.shape[-1]

        h = (jnp.dot(w1c_ref[...], xc, preferred_element_type=jnp.float32)
             + jnp.dot(w1l_ref[...], xl, preferred_element_type=jnp.float32)
             + b1_ref[...])
        h = h * jax.nn.sigmoid(h)

        logits = jnp.dot(w2_ref[...], h, preferred_element_type=jnp.float32) \
            + b2_ref[...]
        m = jnp.max(logits, axis=0, keepdims=True)
        e = jnp.exp(logits - m)
        p = e / jnp.sum(e, axis=0, keepdims=True)

        amax = jnp.argmax(p, axis=0)
        row = jax.lax.broadcasted_iota(jnp.int32, p.shape, 0)
        g = jnp.where(row == amax[None, :], p, 0.0)

        gc = g[0:1, :]
        gl = g[1:2, :]
        keep = ((gc + gl) > 0).astype(jnp.float32)

        oc_ref[0] = xc * gc
        ol_ref[0] = xl * gl
        sl = pl.ds(t * ncols, ncols)
        okeep_ref[b, :, sl] = keep
        oprobs_ref[b, :, sl] = p
        ogate_ref[b, :, sl] = g

        col = jax.lax.broadcasted_iota(jnp.int32, (1, ncols), 1) + t * ncols
        s = jnp.sum(jnp.where(col < hw, keep, 0.0))
        blk = jnp.full((_EPAD, 128), s, dtype=jnp.float32)

        @pl.when(t == 0)
        def _():
            acc_ref[...] = blk

        @pl.when(t != 0)
        def _():
            acc_ref[...] = acc_ref[...] + blk

        @pl.when(t == nt - 1)
        def _():
            oksum_ref[b] = acc_ref[...]

    bufd = pl.Buffered(buffer_count=_INBUF)

    C = zc_hbm.shape[1]
    big_in = pl.BlockSpec((1, C, _TILE), lambda i: (i // nt, 0, i % nt),
                          pipeline_mode=bufd)
    big_out = pl.BlockSpec((1, C, _TILE), lambda i: (i // nt, 0, i % nt))

    pipe = pltpu.emit_pipeline(
        body,
        grid=(zc_hbm.shape[0] * nt,),
        in_specs=[big_in, big_in],
        out_specs=[big_out, big_out],
    )
    pipe(zc_hbm, zl_hbm, oc_hbm, ol_hbm)


@jax.jit
def kernel(z_cam, z_lidar, W1, b1, W2, b2):
    bsz, C, h, w = z_cam.shape
    hw = h * w
    hidden = W1.shape[1]
    E = W2.shape[1]

    zc = z_cam.reshape(bsz, C, hw)
    zl = z_lidar.reshape(bsz, C, hw)
    w1c = W1[:C].T
    w1l = W1[C:].T
    b1c = b1.reshape(hidden, 1)
    w2p = jnp.zeros((_EPAD, hidden), jnp.float32).at[:E].set(W2.T)
    b2p = jnp.full((_EPAD,), _NEG, jnp.float32).at[:E].set(b2).reshape(_EPAD, 1)

    nt = pl.cdiv(hw, _TILE)
    hwp = nt * _TILE

    out_types = (
        jax.ShapeDtypeStruct((bsz, C, hw), jnp.float32),
        jax.ShapeDtypeStruct((bsz, C, hw), jnp.float32),
        jax.ShapeDtypeStruct((bsz, 1, hwp), jnp.float32),
        jax.ShapeDtypeStruct((bsz, _EPAD, hwp), jnp.float32),
        jax.ShapeDtypeStruct((bsz, _EPAD, hwp), jnp.float32),
        jax.ShapeDtypeStruct((bsz, _EPAD, 128), jnp.float32),
    )

    hbm = pl.BlockSpec(memory_space=pltpu.MemorySpace.HBM)
    vmem = pl.BlockSpec(memory_space=pltpu.MemorySpace.VMEM)

    oc, ol, okeep, oprobs, ogate, oksum = pl.pallas_call(
        functools.partial(_outer_kernel, hw, nt),
        in_specs=[hbm, hbm, vmem, vmem, vmem, vmem, vmem],
        out_specs=[hbm, hbm, vmem, vmem, vmem, vmem],
        out_shape=out_types,
        scratch_shapes=[pltpu.VMEM((_EPAD, 128), jnp.float32)],
    )(zc, zl, w1c, w1l, b1c, w2p, b2p)

    zhat_cam = oc.reshape(bsz, C, h, w)
    zhat_lidar = ol.reshape(bsz, C, h, w)
    keep_mask_2d = okeep[:, :, :hw].reshape(bsz, 1, h, w)
    probs = jnp.transpose(oprobs[:, :E, :hw], (0, 2, 1))
    gate = jnp.transpose(ogate[:, :E, :hw], (0, 2, 1))
    keep_ratio = oksum[:, 0:1, 0] / jnp.float32(hw)
    return (zhat_cam, zhat_lidar, keep_mask_2d, probs, gate, keep_ratio)
